# baseline (speedup 1.0000x reference)
import functools

import jax
import jax.numpy as jnp
from jax import lax
from jax.experimental import pallas as pl
from jax.experimental.pallas import tpu as pltpu
from jax.experimental.pallas import tpu_sc as plsc

B = 4096
L = 50
D = 32
NC = 2
NS = 16
NW = NC * NS
BPW = B // NW
G = 16
NG = BPW // G
GL = G * L
CHUNKS = [(0, 128), (128, 128), (256, 128), (384, 128),
          (512, 128), (640, 128), (768, 32)]

_mesh = plsc.VectorSubcoreMesh(core_axis_name="c", subcore_axis_name="s")


@functools.partial(
    pl.kernel,
    out_type=(
        jax.ShapeDtypeStruct((B,), jnp.float32),
        jax.ShapeDtypeStruct((B,), jnp.float32),
        jax.ShapeDtypeStruct((B,), jnp.float32),
    ),
    mesh=_mesh,
    compiler_params=pltpu.CompilerParams(
        use_tc_tiling_on_sc=False,
        needs_layout_passes=False,
    ),
    scratch_types=[
        pltpu.VMEM((GL,), jnp.int32),
        pltpu.VMEM((G,), jnp.int32),
        pltpu.VMEM((GL, D), jnp.float32),
        pltpu.VMEM((G, D), jnp.float32),
        pltpu.VMEM((BPW,), jnp.float32),
        pltpu.SemaphoreType.DMA,
    ],
)
def _sc_affinity(tctx, actx, bctx, tnxt, anxt, bnxt, ttab, atab, btab,
                 tout, aout, bout,
                 cidx, nidx, crows, nrows, outbuf, sem):
    wid = lax.axis_index("s") * NC + lax.axis_index("c")
    lane = lax.iota(jnp.int32, G)

    def one_feature(ctx_hbm, nxt_hbm, tab_hbm, out_hbm):
        def group_body(g, carry):
            b0 = wid * BPW + g * G
            pltpu.sync_copy(ctx_hbm.at[pl.ds(pl.multiple_of(b0 * L, GL), GL)],
                            cidx)
            pltpu.sync_copy(nxt_hbm.at[pl.ds(pl.multiple_of(b0, G), G)], nidx)
            copies = [pltpu.make_async_copy(tab_hbm.at[nidx], nrows, sem)]
            for (s, n) in CHUNKS:
                copies.append(pltpu.make_async_copy(
                    tab_hbm.at[cidx.at[pl.ds(s, n)]],
                    crows.at[pl.ds(s, n)], sem))
            for c in copies:
                c.start()
            for c in copies:
                c.wait()

            def l_body(l, m):
                row_ids = lane * L + l
                acc = jnp.zeros((G,), jnp.float32)
                for d in range(D):
                    dvec = jnp.full((G,), d, jnp.int32)
                    col = plsc.load_gather(crows, [row_ids, dvec])
                    nv = plsc.load_gather(nrows, [lane, dvec])
                    acc = acc + col * nv
                return jnp.maximum(m, acc)

            m = lax.fori_loop(0, L, l_body,
                              jnp.full((G,), -jnp.inf, jnp.float32))
            outbuf[pl.ds(g * G, G)] = m
            return carry

        lax.fori_loop(0, NG, group_body, 0)
        pltpu.sync_copy(outbuf,
                        out_hbm.at[pl.ds(pl.multiple_of(wid * BPW, BPW), BPW)])

    one_feature(tctx, tnxt, ttab, tout)
    one_feature(actx, anxt, atab, aout)
    one_feature(bctx, bnxt, btab, bout)


def kernel(track_context, artist_context, album_context, next_track,
           next_artist, next_album, track_table, artist_table, album_table):
    tctx = track_context.reshape(-1).astype(jnp.int32)
    actx = artist_context.reshape(-1).astype(jnp.int32)
    bctx = album_context.reshape(-1).astype(jnp.int32)
    tnxt = next_track.reshape(-1).astype(jnp.int32)
    anxt = next_artist.reshape(-1).astype(jnp.int32)
    bnxt = next_album.reshape(-1).astype(jnp.int32)
    return _sc_affinity(tctx, actx, bctx, tnxt, anxt, bnxt,
                        track_table, artist_table, album_table)
